# baseline (speedup 1.0000x reference)
import functools

import jax
import jax.numpy as jnp
from jax import lax
from jax.experimental import pallas as pl
from jax.experimental.pallas import tpu as pltpu
from jax.experimental.pallas import tpu_sc as plsc

_N = 256
_NC = 2
_NS = 16
_NW = _NC * _NS
_ROWS = _N * _N
_RPW = _ROWS // _NW
_CHUNK = 128
_NCHUNK = _RPW // _CHUNK
_NBUF = 3


def _sc_gather_body(x_hbm, idx_hbm, out_hbm, idx_v, buf_a, buf_b, buf_c,
                    gsem_a, gsem_b, gsem_c, ssem_a, ssem_b, ssem_c):
    wid = lax.axis_index("s") * _NC + lax.axis_index("c")
    base = wid * _RPW
    pltpu.sync_copy(idx_hbm.at[pl.ds(base, _RPW)], idx_v)

    bufs = (buf_a, buf_b, buf_c)
    gsems = (gsem_a, gsem_b, gsem_c)
    ssems = (ssem_a, ssem_b, ssem_c)

    def gather(c):
        return pltpu.async_copy(
            x_hbm.at[idx_v.at[pl.ds(c * _CHUNK, _CHUNK)]],
            bufs[c % _NBUF], gsems[c % _NBUF])

    def scatter(c):
        return pltpu.async_copy(
            bufs[c % _NBUF], out_hbm.at[pl.ds(base + c * _CHUNK, _CHUNK)],
            ssems[c % _NBUF])

    gs = [None] * _NCHUNK
    ss = [None] * _NCHUNK
    for c in range(min(_NBUF, _NCHUNK)):
        gs[c] = gather(c)
    for c in range(_NCHUNK):
        gs[c].wait()
        ss[c] = scatter(c)
        n = c + _NBUF
        if n < _NCHUNK:
            ss[n - _NBUF].wait()
            gs[n] = gather(n)
    for c in range(max(0, _NCHUNK - _NBUF), _NCHUNK):
        ss[c].wait()


def _sc_gather(x2, idx):
    mesh = plsc.VectorSubcoreMesh(core_axis_name="c", subcore_axis_name="s")
    run = functools.partial(
        pl.kernel, mesh=mesh,
        out_type=jax.ShapeDtypeStruct((_ROWS, _N), jnp.float32),
        scratch_types=[
            pltpu.VMEM((_RPW,), jnp.int32),
            pltpu.VMEM((_CHUNK, _N), jnp.float32),
            pltpu.VMEM((_CHUNK, _N), jnp.float32),
            pltpu.VMEM((_CHUNK, _N), jnp.float32),
            pltpu.SemaphoreType.DMA,
            pltpu.SemaphoreType.DMA,
            pltpu.SemaphoreType.DMA,
            pltpu.SemaphoreType.DMA,
            pltpu.SemaphoreType.DMA,
            pltpu.SemaphoreType.DMA,
        ],
    )(_sc_gather_body)
    return run(x2, idx)


_BP = 8
_TSTEPS = _N // _BP


def _tc_body(y_smem, y_col, x_any, x_seq, out0_any, out2, p_ref, dsem):
    j = pl.program_id(0)

    @pl.when(j == 0)
    def _():
        iota_k = jax.lax.broadcasted_iota(jnp.int32, (_N, _N), 1)
        p_ref[...] = (y_col[...] == iota_k).astype(jnp.float32)

    for b in range(_BP):
        r = j * _BP + b
        src = y_smem[r]
        pltpu.make_async_copy(x_any.at[src], out0_any.at[r], dsem).start()

    xs = x_seq[...].reshape(_BP * _N, _N)
    p = p_ref[...]
    res = jax.lax.dot_general(
        xs, p, (((1,), (1,)), ((), ())),
        preferred_element_type=jnp.float32,
        precision=jax.lax.Precision.DEFAULT)
    out2[...] = res.reshape(_BP, _N, _N)

    @pl.when(j > 0)
    def _():
        for b in range(_BP):
            pltpu.make_async_copy(x_any.at[0], out0_any.at[0], dsem).wait()

    @pl.when(j == _TSTEPS - 1)
    def _():
        for b in range(_BP):
            pltpu.make_async_copy(x_any.at[0], out0_any.at[0], dsem).wait()


def _tc_part(x, y32):
    y_col = y32.reshape(_N, 1)
    grid_spec = pltpu.PrefetchScalarGridSpec(
        num_scalar_prefetch=1,
        grid=(_TSTEPS,),
        in_specs=[
            pl.BlockSpec((_N, 1), lambda j, y_ref: (0, 0)),
            pl.BlockSpec(memory_space=pl.ANY),
            pl.BlockSpec((_BP, _N, _N), lambda j, y_ref: (j, 0, 0)),
        ],
        out_specs=[
            pl.BlockSpec(memory_space=pl.ANY),
            pl.BlockSpec((_BP, _N, _N), lambda j, y_ref: (j, 0, 0)),
        ],
        scratch_shapes=[pltpu.VMEM((_N, _N), jnp.float32),
                        pltpu.SemaphoreType.DMA],
    )
    out_shape = [jax.ShapeDtypeStruct((_N, _N, _N), jnp.float32),
                 jax.ShapeDtypeStruct((_N, _N, _N), jnp.float32)]
    return pl.pallas_call(
        _tc_body, grid_spec=grid_spec, out_shape=out_shape,
    )(y32, y_col, x, x)


def kernel(x, y):
    y32 = y.astype(jnp.int32)
    ar = jnp.arange(_N, dtype=jnp.int32)
    idx1 = (ar[:, None] * _N + y32[None, :]).reshape(-1)
    x2 = x.reshape(_ROWS, _N)
    out1 = _sc_gather(x2, idx1).reshape(_N, _N, _N)
    out0, out2 = _tc_part(x, y32)
    return (out0, out1, out2)


# speedup vs baseline: 13.0687x; 13.0687x over previous
import functools

import jax
import jax.numpy as jnp
from jax import lax
from jax.experimental import pallas as pl
from jax.experimental.pallas import tpu as pltpu
from jax.experimental.pallas import tpu_sc as plsc

_N = 256
_NC = 2
_NS = 16
_NW = _NC * _NS
_ROWS = _N * _N
_RPW = _ROWS // _NW
_CHUNK = 128
_NCHUNK = _RPW // _CHUNK
_NBUF = 3


def _sc_gather_body(x_hbm, idx0_hbm, idx1_hbm, out0_hbm, out1_hbm,
                    idx0_v, idx1_v, buf_a, buf_b, buf_c,
                    gsem_a, gsem_b, gsem_c, ssem_a, ssem_b, ssem_c):
    wid = lax.axis_index("s") * _NC + lax.axis_index("c")
    base = wid * _RPW
    pltpu.sync_copy(idx0_hbm.at[pl.ds(base, _RPW)], idx0_v)
    pltpu.sync_copy(idx1_hbm.at[pl.ds(base, _RPW)], idx1_v)

    bufs = (buf_a, buf_b, buf_c)
    gsems = (gsem_a, gsem_b, gsem_c)
    ssems = (ssem_a, ssem_b, ssem_c)
    total = 2 * _NCHUNK

    def gather(c):
        if c < _NCHUNK:
            src = idx0_v.at[pl.ds(c * _CHUNK, _CHUNK)]
        else:
            src = idx1_v.at[pl.ds((c - _NCHUNK) * _CHUNK, _CHUNK)]
        return pltpu.async_copy(x_hbm.at[src], bufs[c % _NBUF],
                                gsems[c % _NBUF])

    def scatter(c):
        if c < _NCHUNK:
            dst = out0_hbm.at[pl.ds(base + c * _CHUNK, _CHUNK)]
        else:
            dst = out1_hbm.at[pl.ds(base + (c - _NCHUNK) * _CHUNK, _CHUNK)]
        return pltpu.async_copy(bufs[c % _NBUF], dst, ssems[c % _NBUF])

    gs = [None] * total
    ss = [None] * total
    for c in range(min(_NBUF, total)):
        gs[c] = gather(c)
    for c in range(total):
        gs[c].wait()
        ss[c] = scatter(c)
        n = c + _NBUF
        if n < total:
            ss[n - _NBUF].wait()
            gs[n] = gather(n)
    for c in range(max(0, total - _NBUF), total):
        ss[c].wait()


def _sc_gather2(x2, idx0, idx1):
    mesh = plsc.VectorSubcoreMesh(core_axis_name="c", subcore_axis_name="s")
    run = functools.partial(
        pl.kernel, mesh=mesh,
        out_type=[jax.ShapeDtypeStruct((_ROWS, _N), jnp.float32),
                  jax.ShapeDtypeStruct((_ROWS, _N), jnp.float32)],
        scratch_types=[
            pltpu.VMEM((_RPW,), jnp.int32),
            pltpu.VMEM((_RPW,), jnp.int32),
            pltpu.VMEM((_CHUNK, _N), jnp.float32),
            pltpu.VMEM((_CHUNK, _N), jnp.float32),
            pltpu.VMEM((_CHUNK, _N), jnp.float32),
            pltpu.SemaphoreType.DMA,
            pltpu.SemaphoreType.DMA,
            pltpu.SemaphoreType.DMA,
            pltpu.SemaphoreType.DMA,
            pltpu.SemaphoreType.DMA,
            pltpu.SemaphoreType.DMA,
        ],
    )(_sc_gather_body)
    return run(x2, idx0, idx1)


_BP = 16
_TSTEPS = _N // _BP


def _tc_body(y_smem, y_col, x_seq, out2, p_ref):
    j = pl.program_id(0)

    @pl.when(j == 0)
    def _():
        iota_k = jax.lax.broadcasted_iota(jnp.int32, (_N, _N), 1)
        p_ref[...] = (y_col[...] == iota_k).astype(jnp.float32)

    xs = x_seq[...].reshape(_BP * _N, _N)
    p = p_ref[...]
    res = jax.lax.dot_general(
        xs, p, (((1,), (1,)), ((), ())),
        preferred_element_type=jnp.float32,
        precision=jax.lax.Precision.DEFAULT)
    out2[...] = res.reshape(_BP, _N, _N)


def _tc_matmuls(x, y32):
    y_col = y32.reshape(_N, 1)
    grid_spec = pltpu.PrefetchScalarGridSpec(
        num_scalar_prefetch=1,
        grid=(_TSTEPS,),
        in_specs=[
            pl.BlockSpec((_N, 1), lambda j, y_ref: (0, 0)),
            pl.BlockSpec((_BP, _N, _N), lambda j, y_ref: (j, 0, 0)),
        ],
        out_specs=pl.BlockSpec((_BP, _N, _N), lambda j, y_ref: (j, 0, 0)),
        scratch_shapes=[pltpu.VMEM((_N, _N), jnp.float32)],
    )
    return pl.pallas_call(
        _tc_body, grid_spec=grid_spec,
        out_shape=jax.ShapeDtypeStruct((_N, _N, _N), jnp.float32),
    )(y32, y_col, x)


def kernel(x, y):
    y32 = y.astype(jnp.int32)
    ar = jnp.arange(_N, dtype=jnp.int32)
    idx0 = (y32[:, None] * _N + ar[None, :]).reshape(-1)
    idx1 = (ar[:, None] * _N + y32[None, :]).reshape(-1)
    x2 = x.reshape(_ROWS, _N)
    out0_2, out1_2 = _sc_gather2(x2, idx0, idx1)
    out0 = out0_2.reshape(_N, _N, _N)
    out1 = out1_2.reshape(_N, _N, _N)
    out2 = _tc_matmuls(x, y32)
    return (out0, out1, out2)


# speedup vs baseline: 13.1400x; 1.0055x over previous
import functools

import jax
import jax.numpy as jnp
from jax import lax
from jax.experimental import pallas as pl
from jax.experimental.pallas import tpu as pltpu
from jax.experimental.pallas import tpu_sc as plsc

_N = 256
_NC = 2
_NS = 16
_NW = _NC * _NS
_ROWS = _N * _N
_RPW = _ROWS // _NW
_CHUNK = 128
_NCHUNK = _RPW // _CHUNK
_NBUF = 3


_CHUNK = 64
_NCHUNK = _RPW // _CHUNK
_NBUF = 6


def _sc_gather_body(x2_hbm, idx0_hbm, idx1_hbm, out0_hbm, out1_hbm, *scr):
    idx0_v, idx1_v = scr[0], scr[1]
    bufs = scr[2:2 + _NBUF]
    gsems = scr[2 + _NBUF:2 + 2 * _NBUF]
    ssems = scr[2 + 2 * _NBUF:2 + 3 * _NBUF]
    wid = lax.axis_index("s") * _NC + lax.axis_index("c")
    base = wid * _RPW
    pltpu.sync_copy(idx0_hbm.at[pl.ds(base, _RPW)], idx0_v)
    pltpu.sync_copy(idx1_hbm.at[pl.ds(base, _RPW)], idx1_v)

    total = 2 * _NCHUNK

    def gather(c, buf, sem):
        if c < _NCHUNK:
            src = idx0_v.at[pl.ds(c * _CHUNK, _CHUNK)]
        else:
            src = idx1_v.at[pl.ds((c - _NCHUNK) * _CHUNK, _CHUNK)]
        return pltpu.async_copy(x2_hbm.at[src], buf, sem)

    def scatter(c, buf, sem):
        if c < _NCHUNK:
            dst = out0_hbm.at[pl.ds(base + c * _CHUNK, _CHUNK)]
        else:
            dst = out1_hbm.at[pl.ds(base + (c - _NCHUNK) * _CHUNK, _CHUNK)]
        return pltpu.async_copy(buf, dst, sem)

    gs = [None] * total
    ss = [None] * total
    for c in range(min(_NBUF, total)):
        gs[c] = gather(c, bufs[c % _NBUF], gsems[c % _NBUF])
    for c in range(total):
        gs[c].wait()
        ss[c] = scatter(c, bufs[c % _NBUF], ssems[c % _NBUF])
        n = c + _NBUF
        if n < total:
            ss[n - _NBUF].wait()
            gs[n] = gather(n, bufs[n % _NBUF], gsems[n % _NBUF])
    for c in range(max(0, total - _NBUF), total):
        ss[c].wait()


def _sc_gather2(x2, idx0, idx1):
    mesh = plsc.VectorSubcoreMesh(core_axis_name="c", subcore_axis_name="s")
    dma = pltpu.SemaphoreType.DMA
    run = functools.partial(
        pl.kernel, mesh=mesh,
        out_type=[jax.ShapeDtypeStruct((_ROWS, _N), jnp.float32),
                  jax.ShapeDtypeStruct((_ROWS, _N), jnp.float32)],
        scratch_types=[
            pltpu.VMEM((_RPW,), jnp.int32),
            pltpu.VMEM((_RPW,), jnp.int32),
        ] + [pltpu.VMEM((_CHUNK, _N), jnp.float32)] * _NBUF
          + [dma] * (2 * _NBUF),
    )(_sc_gather_body)
    return run(x2, idx0, idx1)


_BP = 16
_TSTEPS = _N // _BP


def _tc_body(y_smem, y_col, x_seq, out2, p_ref):
    j = pl.program_id(0)

    @pl.when(j == 0)
    def _():
        iota_k = jax.lax.broadcasted_iota(jnp.int32, (_N, _N), 1)
        p_ref[...] = (y_col[...] == iota_k).astype(jnp.float32)

    xs = x_seq[...].reshape(_BP * _N, _N)
    p = p_ref[...]
    res = jax.lax.dot_general(
        xs, p, (((1,), (1,)), ((), ())),
        preferred_element_type=jnp.float32,
        precision=jax.lax.Precision.DEFAULT)
    out2[...] = res.reshape(_BP, _N, _N)


def _tc_matmuls(x, y32):
    y_col = y32.reshape(_N, 1)
    grid_spec = pltpu.PrefetchScalarGridSpec(
        num_scalar_prefetch=1,
        grid=(_TSTEPS,),
        in_specs=[
            pl.BlockSpec((_N, 1), lambda j, y_ref: (0, 0)),
            pl.BlockSpec((_BP, _N, _N), lambda j, y_ref: (j, 0, 0)),
        ],
        out_specs=pl.BlockSpec((_BP, _N, _N), lambda j, y_ref: (j, 0, 0)),
        scratch_shapes=[pltpu.VMEM((_N, _N), jnp.float32)],
    )
    return pl.pallas_call(
        _tc_body, grid_spec=grid_spec,
        out_shape=jax.ShapeDtypeStruct((_N, _N, _N), jnp.float32),
    )(y32, y_col, x)


def kernel(x, y):
    y32 = y.astype(jnp.int32)
    ar = jnp.arange(_N, dtype=jnp.int32)
    idx0 = (y32[:, None] * _N + ar[None, :]).reshape(-1)
    idx1 = (ar[:, None] * _N + y32[None, :]).reshape(-1)
    x2 = x.reshape(_ROWS, _N)
    out0_2, out1_2 = _sc_gather2(x2, idx0, idx1)
    out0 = out0_2.reshape(_N, _N, _N)
    out1 = out1_2.reshape(_N, _N, _N)
    out2 = _tc_matmuls(x, y32)
    return (out0, out1, out2)
